# baseline (speedup 1.0000x reference)
import functools

import jax
import jax.numpy as jnp
from jax import lax
from jax.experimental import pallas as pl
from jax.experimental.pallas import tpu as pltpu
from jax.experimental.pallas import tpu_sc as plsc

_L = 16
_NW = 32
_SLAB = 128


def _rsqrt(x):
    i = lax.bitcast_convert_type(x, jnp.int32)
    y = lax.bitcast_convert_type(0x5F3759DF - (i >> 1), jnp.float32)
    xh = 0.5 * x
    y = y * (1.5 - xh * y * y)
    y = y * (1.5 - xh * y * y)
    return y


def _sqrt(x):
    return x * _rsqrt(x)


def _atan2(y, x, x_nonneg=False):
    ax = jnp.abs(x)
    ay = jnp.abs(y)
    hi = jnp.maximum(ax, ay)
    lo = jnp.minimum(ax, ay)
    c = lo > 0.41421356 * hi
    num = jnp.where(c, lo - hi, lo)
    den = jnp.where(c, lo + hi, hi)
    t = num / jnp.maximum(den, 1e-30)
    z = t * t
    p = (
        ((8.05374449538e-2 * z - 1.38776856032e-1) * z + 1.99777106478e-1) * z
        - 3.33329491539e-1
    ) * z * t + t
    r = jnp.where(c, p + 0.78539816339745, p)
    r = jnp.where(ay > ax, 1.5707963267949 - r, r)
    if not x_nonneg:
        r = jnp.where(x < 0.0, 3.14159265358979 - r, r)
    return jnp.where(y < 0.0, -r, r)


def _make_sc_kernel(n, k):
    full_slabs = n // _SLAB
    tail_pts = n % _SLAB
    assert tail_pts % _L == 0
    slabs = full_slabs + (1 if tail_pts else 0)
    j_iters = (slabs + _NW - 1) // _NW
    stage = n + _L

    mesh = plsc.VectorSubcoreMesh(core_axis_name="c", subcore_axis_name="s")
    out_type = (
        jax.ShapeDtypeStruct((9 * k, n), jnp.float32),
        jax.ShapeDtypeStruct((k, 1, n), jnp.float32),
        jax.ShapeDtypeStruct((n,), jnp.float32),
        jax.ShapeDtypeStruct((9 * k * max(tail_pts, _L) // _SLAB, _SLAB), jnp.float32),
        jax.ShapeDtypeStruct((k * max(tail_pts, _L) // _SLAB, _SLAB), jnp.float32),
    )
    scratch_types = [
        pltpu.VMEM((stage,), jnp.float32),
        pltpu.VMEM((stage,), jnp.float32),
        pltpu.VMEM((stage,), jnp.float32),
        pltpu.VMEM((_SLAB, k), jnp.int32),
        pltpu.VMEM((9 * k, _SLAB), jnp.float32),
        pltpu.VMEM((k, _SLAB), jnp.float32),
        pltpu.VMEM((_SLAB,), jnp.float32),
        pltpu.VMEM((2, _SLAB), jnp.float32),
        pltpu.SemaphoreType.DMA,
    ]

    @functools.partial(
        pl.kernel,
        out_type=out_type,
        mesh=mesh,
        scratch_types=scratch_types,
        compiler_params=pltpu.CompilerParams(needs_layout_passes=False),
    )
    def sc_kernel(cols_hbm, idx_hbm, o_rep, o_exp, o_lg, o_rep_t, o_exp_t,
                  x_s, y_s, z_s, idx_s, rep_s, exp_s, lg_s, dir_s, sem):
        w = lax.axis_index("s") * 2 + lax.axis_index("c")
        pltpu.sync_copy(cols_hbm.at[pl.ds(0, stage)], x_s)
        pltpu.sync_copy(cols_hbm.at[pl.ds(stage, stage)], y_s)
        pltpu.sync_copy(cols_hbm.at[pl.ds(2 * stage, stage)], z_s)

        lane = lax.iota(jnp.int32, _L)

        def do_slab(t, npts):
            is_tail = npts != _SLAB
            sbase = t * _SLAB
            pltpu.sync_copy(idx_hbm.at[pl.ds(sbase, npts)], idx_s.at[pl.ds(0, npts), :])

            def st_rep(row, po, val):
                if is_tail:
                    a = row * npts + po
                    rep_s[a >> 7, pl.ds(a & (_SLAB - 1), _L)] = val
                else:
                    rep_s[row, pl.ds(po, _L)] = val

            def ld_rep(row, po):
                if is_tail:
                    a = row * npts + po
                    return rep_s[a >> 7, pl.ds(a & (_SLAB - 1), _L)]
                return rep_s[row, pl.ds(po, _L)]

            def st_exp(row, po, val):
                if is_tail:
                    a = row * npts + po
                    exp_s[a >> 7, pl.ds(a & (_SLAB - 1), _L)] = val
                else:
                    exp_s[row, pl.ds(po, _L)] = val

            for pg in range(npts // _L):
                base = sbase + pg * _L
                po = pg * _L
                xi = x_s[pl.ds(base, _L)]
                yi = y_s[pl.ds(base, _L)]
                zi = z_s[pl.ds(base, _L)]
                lane_po = lane + po

                def pass1(kk, carry):
                    maxd, sx, sy, sz = carry
                    iv = plsc.load_gather(idx_s, [lane_po, jnp.full((_L,), kk, jnp.int32)])
                    nx = plsc.load_gather(x_s, [iv])
                    ny = plsc.load_gather(y_s, [iv])
                    nz = plsc.load_gather(z_s, [iv])
                    rx = xi - nx
                    ry = yi - ny
                    rz = zi - nz
                    xy2 = rx * rx + ry * ry
                    d2 = xy2 + rz * rz
                    xyd = _sqrt(xy2)
                    dis = _sqrt(d2)
                    alpha = _atan2(ry, rx)
                    beta = _atan2(rz, xyd, x_nonneg=True)
                    st_rep(kk, po, alpha)
                    st_rep(k + kk, po, beta)
                    st_rep(2 * k + kk, po, dis)
                    st_rep(3 * k + kk, po, xi)
                    st_rep(4 * k + kk, po, yi)
                    st_rep(5 * k + kk, po, zi)
                    st_rep(6 * k + kk, po, nx)
                    st_rep(7 * k + kk, po, ny)
                    st_rep(8 * k + kk, po, nz)
                    st_exp(kk, po, jnp.exp(-dis))
                    return (
                        jnp.maximum(maxd, dis),
                        sx + nx,
                        sy + ny,
                        sz + nz,
                    )

                zero = jnp.zeros((_L,), jnp.float32)
                maxd, sx, sy, sz = plsc.parallel_loop(
                    0, k, unroll=2, carry=(zero, zero, zero, zero)
                )(pass1)

                inv_k = 1.0 / k
                dx = xi - sx * inv_k
                dy = yi - sy * inv_k
                dz = zi - sz * inv_k
                dal = _atan2(dy, dx)
                dbe = _atan2(dz, _sqrt(dx * dx + dy * dy), x_nonneg=True)
                gd = _sqrt(xi * xi + yi * yi + zi * zi)
                lg_s[pl.ds(po, _L)] = maxd * maxd * maxd / gd
                dir_s[0, pl.ds(po, _L)] = dal
                dir_s[1, pl.ds(po, _L)] = dbe

            if not is_tail:
                h1 = pltpu.async_copy(
                    rep_s.at[pl.ds(2 * k, 7 * k), :],
                    o_rep.at[pl.ds(2 * k, 7 * k), pl.ds(sbase, _SLAB)],
                    sem,
                )
                h2 = pltpu.async_copy(
                    exp_s, o_exp.at[:, 0, pl.ds(sbase, _SLAB)], sem
                )
                h3 = pltpu.async_copy(lg_s, o_lg.at[pl.ds(sbase, _SLAB)], sem)

            for pg in range(npts // _L):
                po = pg * _L
                dal = dir_s[0, pl.ds(po, _L)]
                dbe = dir_s[1, pl.ds(po, _L)]

                @plsc.parallel_loop(0, k, unroll=4)
                def pass2(kk):
                    st_rep(kk, po, ld_rep(kk, po) - dal)
                    st_rep(k + kk, po, ld_rep(k + kk, po) - dbe)

            if not is_tail:
                h4 = pltpu.async_copy(
                    rep_s.at[pl.ds(0, 2 * k), :],
                    o_rep.at[pl.ds(0, 2 * k), pl.ds(sbase, _SLAB)],
                    sem,
                )
                h1.wait()
                h2.wait()
                h3.wait()
                h4.wait()
            else:
                nrows_r = 9 * k * npts // _SLAB
                nrows_e = k * npts // _SLAB
                pltpu.sync_copy(rep_s.at[pl.ds(0, nrows_r), :], o_rep_t)
                pltpu.sync_copy(exp_s.at[pl.ds(0, nrows_e), :], o_exp_t)
                pltpu.sync_copy(
                    lg_s.at[pl.ds(0, tail_pts)], o_lg.at[pl.ds(sbase, tail_pts)]
                )

        def slab_body(j, _):
            t = j * _NW + w

            @pl.when(t < full_slabs)
            def _():
                do_slab(t, _SLAB)

            if tail_pts:
                @pl.when(t == full_slabs)
                def _():
                    do_slab(t, tail_pts)

            return 0

        lax.fori_loop(0, j_iters, slab_body, 0)

    return sc_kernel


def kernel(xyz, neigh_idx, lengths):
    del lengths
    n, k = neigh_idx.shape
    stage = n + _L
    cols = jnp.zeros((3, stage), jnp.float32).at[:, :n].set(xyz.T).reshape(-1)
    rep, expd, lg, rep_t, exp_t = _make_sc_kernel(n, k)(cols, neigh_idx)
    big = rep.reshape(9, k, n).transpose(2, 1, 0)
    big_e = expd.transpose(2, 0, 1)
    tail = n % _SLAB
    if tail:
        nb = n - tail
        tail_rep = rep_t.reshape(9, k, tail).transpose(2, 1, 0)
        tail_exp = exp_t.reshape(k, tail).transpose(1, 0).reshape(tail, k, 1)
        big = lax.dynamic_update_slice(big, tail_rep, (nb, 0, 0))
        big_e = lax.dynamic_update_slice(big_e, tail_exp, (nb, 0, 0))
    return (big, big_e, lg.reshape(n, 1))


# speedup vs baseline: 1.0294x; 1.0294x over previous
import functools

import jax
import jax.numpy as jnp
from jax import lax
from jax.experimental import pallas as pl
from jax.experimental.pallas import tpu as pltpu
from jax.experimental.pallas import tpu_sc as plsc

_L = 16
_NW = 32
_SLAB = 128


def _rsqrt(x):
    i = lax.bitcast_convert_type(x, jnp.int32)
    y = lax.bitcast_convert_type(0x5F3759DF - (i >> 1), jnp.float32)
    xh = 0.5 * x
    y = y * (1.5 - xh * y * y)
    y = y * (1.5 - xh * y * y)
    return y


def _sqrt(x):
    return x * _rsqrt(x)


def _atan2(y, x, x_nonneg=False):
    ax = jnp.abs(x)
    ay = jnp.abs(y)
    hi = jnp.maximum(ax, ay)
    lo = jnp.minimum(ax, ay)
    c = lo > 0.41421356 * hi
    num = jnp.where(c, lo - hi, lo)
    den = jnp.where(c, lo + hi, hi)
    t = num / jnp.maximum(den, 1e-30)
    z = t * t
    p = (
        ((8.05374449538e-2 * z - 1.38776856032e-1) * z + 1.99777106478e-1) * z
        - 3.33329491539e-1
    ) * z * t + t
    r = jnp.where(c, p + 0.78539816339745, p)
    r = jnp.where(ay > ax, 1.5707963267949 - r, r)
    if not x_nonneg:
        r = jnp.where(x < 0.0, 3.14159265358979 - r, r)
    return jnp.where(y < 0.0, -r, r)


def _make_sc_kernel(n, k):
    full_slabs = n // _SLAB
    tail_pts = n % _SLAB
    assert tail_pts % _L == 0
    slabs = full_slabs + (1 if tail_pts else 0)
    j_iters = (slabs + _NW - 1) // _NW
    stage = n + _L

    mesh = plsc.VectorSubcoreMesh(core_axis_name="c", subcore_axis_name="s")
    out_type = (
        jax.ShapeDtypeStruct((9 * k, n), jnp.float32),
        jax.ShapeDtypeStruct((k, 1, n), jnp.float32),
        jax.ShapeDtypeStruct((n,), jnp.float32),
        jax.ShapeDtypeStruct((9 * k * max(tail_pts, _L) // _SLAB, _SLAB), jnp.float32),
        jax.ShapeDtypeStruct((k * max(tail_pts, _L) // _SLAB, _SLAB), jnp.float32),
    )
    scratch_types = [
        pltpu.VMEM((stage,), jnp.float32),
        pltpu.VMEM((stage,), jnp.float32),
        pltpu.VMEM((stage,), jnp.float32),
        pltpu.VMEM((_SLAB, k), jnp.int32),
        pltpu.VMEM((9 * k, _SLAB), jnp.float32),
        pltpu.VMEM((k, _SLAB), jnp.float32),
        pltpu.VMEM((_SLAB,), jnp.float32),
        pltpu.VMEM((2, _SLAB), jnp.float32),
        pltpu.SemaphoreType.DMA,
    ]

    @functools.partial(
        pl.kernel,
        out_type=out_type,
        mesh=mesh,
        scratch_types=scratch_types,
        compiler_params=pltpu.CompilerParams(needs_layout_passes=False),
    )
    def sc_kernel(cols_hbm, idx_hbm, o_rep, o_exp, o_lg, o_rep_t, o_exp_t,
                  x_s, y_s, z_s, idx_s, rep_s, exp_s, lg_s, dir_s, sem):
        w = lax.axis_index("s") * 2 + lax.axis_index("c")
        pltpu.sync_copy(cols_hbm.at[pl.ds(0, stage)], x_s)
        pltpu.sync_copy(cols_hbm.at[pl.ds(stage, stage)], y_s)
        pltpu.sync_copy(cols_hbm.at[pl.ds(2 * stage, stage)], z_s)

        lane = lax.iota(jnp.int32, _L)

        def do_slab(t, npts):
            is_tail = npts != _SLAB
            sbase = t * _SLAB
            pltpu.sync_copy(idx_hbm.at[pl.ds(sbase, npts)], idx_s.at[pl.ds(0, npts), :])

            def st_rep(row, po, val):
                if is_tail:
                    a = row * npts + po
                    rep_s[a >> 7, pl.ds(a & (_SLAB - 1), _L)] = val
                else:
                    rep_s[row, pl.ds(po, _L)] = val

            def ld_rep(row, po):
                if is_tail:
                    a = row * npts + po
                    return rep_s[a >> 7, pl.ds(a & (_SLAB - 1), _L)]
                return rep_s[row, pl.ds(po, _L)]

            def st_exp(row, po, val):
                if is_tail:
                    a = row * npts + po
                    exp_s[a >> 7, pl.ds(a & (_SLAB - 1), _L)] = val
                else:
                    exp_s[row, pl.ds(po, _L)] = val

            for pg in range(npts // _L):
                base = sbase + pg * _L
                po = pg * _L
                xi = x_s[pl.ds(base, _L)]
                yi = y_s[pl.ds(base, _L)]
                zi = z_s[pl.ds(base, _L)]
                lane_po = lane + po

                def pass1(kk, carry):
                    maxd, sx, sy, sz = carry
                    iv = plsc.load_gather(idx_s, [lane_po, jnp.full((_L,), kk, jnp.int32)])
                    nx = plsc.load_gather(x_s, [iv])
                    ny = plsc.load_gather(y_s, [iv])
                    nz = plsc.load_gather(z_s, [iv])
                    rx = xi - nx
                    ry = yi - ny
                    rz = zi - nz
                    xy2 = rx * rx + ry * ry
                    d2 = xy2 + rz * rz
                    xyd = _sqrt(xy2)
                    dis = _sqrt(d2)
                    alpha = _atan2(ry, rx)
                    beta = _atan2(rz, xyd, x_nonneg=True)
                    st_rep(kk, po, alpha)
                    st_rep(k + kk, po, beta)
                    st_rep(2 * k + kk, po, dis)
                    st_rep(3 * k + kk, po, xi)
                    st_rep(4 * k + kk, po, yi)
                    st_rep(5 * k + kk, po, zi)
                    st_rep(6 * k + kk, po, nx)
                    st_rep(7 * k + kk, po, ny)
                    st_rep(8 * k + kk, po, nz)
                    st_exp(kk, po, jnp.exp(-dis))
                    return (
                        jnp.maximum(maxd, dis),
                        sx + nx,
                        sy + ny,
                        sz + nz,
                    )

                zero = jnp.zeros((_L,), jnp.float32)
                maxd, sx, sy, sz = plsc.parallel_loop(
                    0, k, unroll=3, carry=(zero, zero, zero, zero)
                )(pass1)

                inv_k = 1.0 / k
                dx = xi - sx * inv_k
                dy = yi - sy * inv_k
                dz = zi - sz * inv_k
                dal = _atan2(dy, dx)
                dbe = _atan2(dz, _sqrt(dx * dx + dy * dy), x_nonneg=True)
                gd = _sqrt(xi * xi + yi * yi + zi * zi)
                lg_s[pl.ds(po, _L)] = maxd * maxd * maxd / gd
                dir_s[0, pl.ds(po, _L)] = dal
                dir_s[1, pl.ds(po, _L)] = dbe

            if not is_tail:
                h1 = pltpu.async_copy(
                    rep_s.at[pl.ds(2 * k, 7 * k), :],
                    o_rep.at[pl.ds(2 * k, 7 * k), pl.ds(sbase, _SLAB)],
                    sem,
                )
                h2 = pltpu.async_copy(
                    exp_s, o_exp.at[:, 0, pl.ds(sbase, _SLAB)], sem
                )
                h3 = pltpu.async_copy(lg_s, o_lg.at[pl.ds(sbase, _SLAB)], sem)

            for pg in range(npts // _L):
                po = pg * _L
                dal = dir_s[0, pl.ds(po, _L)]
                dbe = dir_s[1, pl.ds(po, _L)]

                @plsc.parallel_loop(0, k, unroll=2)
                def pass2(kk):
                    st_rep(kk, po, ld_rep(kk, po) - dal)
                    st_rep(k + kk, po, ld_rep(k + kk, po) - dbe)

            if not is_tail:
                h4 = pltpu.async_copy(
                    rep_s.at[pl.ds(0, 2 * k), :],
                    o_rep.at[pl.ds(0, 2 * k), pl.ds(sbase, _SLAB)],
                    sem,
                )
                h1.wait()
                h2.wait()
                h3.wait()
                h4.wait()
            else:
                nrows_r = 9 * k * npts // _SLAB
                nrows_e = k * npts // _SLAB
                pltpu.sync_copy(rep_s.at[pl.ds(0, nrows_r), :], o_rep_t)
                pltpu.sync_copy(exp_s.at[pl.ds(0, nrows_e), :], o_exp_t)
                pltpu.sync_copy(
                    lg_s.at[pl.ds(0, tail_pts)], o_lg.at[pl.ds(sbase, tail_pts)]
                )

        def slab_body(j, _):
            t = j * _NW + w

            @pl.when(t < full_slabs)
            def _():
                do_slab(t, _SLAB)

            if tail_pts:
                @pl.when(t == full_slabs)
                def _():
                    do_slab(t, tail_pts)

            return 0

        lax.fori_loop(0, j_iters, slab_body, 0)

    return sc_kernel


def kernel(xyz, neigh_idx, lengths):
    del lengths
    n, k = neigh_idx.shape
    stage = n + _L
    cols = jnp.zeros((3, stage), jnp.float32).at[:, :n].set(xyz.T).reshape(-1)
    rep, expd, lg, rep_t, exp_t = _make_sc_kernel(n, k)(cols, neigh_idx)
    big = rep.reshape(9, k, n).transpose(2, 1, 0)
    big_e = expd.transpose(2, 0, 1)
    tail = n % _SLAB
    if tail:
        nb = n - tail
        tail_rep = rep_t.reshape(9, k, tail).transpose(2, 1, 0)
        tail_exp = exp_t.reshape(k, tail).transpose(1, 0).reshape(tail, k, 1)
        big = lax.dynamic_update_slice(big, tail_rep, (nb, 0, 0))
        big_e = lax.dynamic_update_slice(big_e, tail_exp, (nb, 0, 0))
    return (big, big_e, lg.reshape(n, 1))


# speedup vs baseline: 1.1057x; 1.0742x over previous
import functools

import jax
import jax.numpy as jnp
from jax import lax
from jax.experimental import pallas as pl
from jax.experimental.pallas import tpu as pltpu
from jax.experimental.pallas import tpu_sc as plsc

_L = 16
_NW = 32
_SLAB = 128


def _rsqrt(x):
    i = lax.bitcast_convert_type(x, jnp.int32)
    y = lax.bitcast_convert_type(0x5F3759DF - (i >> 1), jnp.float32)
    xh = 0.5 * x
    y = y * (1.5 - xh * y * y)
    y = y * (1.5 - xh * y * y)
    return y


def _sqrt(x):
    return x * _rsqrt(x)


def _atan2(y, x, x_nonneg=False):
    ax = jnp.abs(x)
    ay = jnp.abs(y)
    hi = jnp.maximum(ax, ay)
    lo = jnp.minimum(ax, ay)
    t = lo / jnp.maximum(hi, 1e-30)
    z = t * t
    p = t * (
        0.99997726
        + z
        * (
            -0.33262347
            + z
            * (0.19354346 + z * (-0.11643287 + z * (0.05265332 + z * -0.01172120)))
        )
    )
    r = jnp.where(ay > ax, 1.5707963267949 - p, p)
    if not x_nonneg:
        r = jnp.where(x < 0.0, 3.14159265358979 - r, r)
    return jnp.where(y < 0.0, -r, r)


def _make_sc_kernel(n, k):
    full_slabs = n // _SLAB
    tail_pts = n % _SLAB
    assert tail_pts % _L == 0
    slabs = full_slabs + (1 if tail_pts else 0)
    j_iters = (slabs + _NW - 1) // _NW
    stage = n + _L

    mesh = plsc.VectorSubcoreMesh(core_axis_name="c", subcore_axis_name="s")
    out_type = (
        jax.ShapeDtypeStruct((9 * k, n), jnp.float32),
        jax.ShapeDtypeStruct((k, 1, n), jnp.float32),
        jax.ShapeDtypeStruct((n,), jnp.float32),
        jax.ShapeDtypeStruct((9 * k * max(tail_pts, _L) // _SLAB, _SLAB), jnp.float32),
        jax.ShapeDtypeStruct((k * max(tail_pts, _L) // _SLAB, _SLAB), jnp.float32),
    )
    scratch_types = [
        pltpu.VMEM((stage,), jnp.float32),
        pltpu.VMEM((stage,), jnp.float32),
        pltpu.VMEM((stage,), jnp.float32),
        pltpu.VMEM((_SLAB, k), jnp.int32),
        pltpu.VMEM((9 * k, _SLAB), jnp.float32),
        pltpu.VMEM((k, _SLAB), jnp.float32),
        pltpu.VMEM((_SLAB,), jnp.float32),
        pltpu.VMEM((2, _SLAB), jnp.float32),
        pltpu.SemaphoreType.DMA,
    ]

    @functools.partial(
        pl.kernel,
        out_type=out_type,
        mesh=mesh,
        scratch_types=scratch_types,
        compiler_params=pltpu.CompilerParams(needs_layout_passes=False),
    )
    def sc_kernel(cols_hbm, idx_hbm, o_rep, o_exp, o_lg, o_rep_t, o_exp_t,
                  x_s, y_s, z_s, idx_s, rep_s, exp_s, lg_s, dir_s, sem):
        w = lax.axis_index("s") * 2 + lax.axis_index("c")
        pltpu.sync_copy(cols_hbm.at[pl.ds(0, stage)], x_s)
        pltpu.sync_copy(cols_hbm.at[pl.ds(stage, stage)], y_s)
        pltpu.sync_copy(cols_hbm.at[pl.ds(2 * stage, stage)], z_s)

        lane = lax.iota(jnp.int32, _L)

        def do_slab(t, npts):
            is_tail = npts != _SLAB
            sbase = t * _SLAB
            pltpu.sync_copy(idx_hbm.at[pl.ds(sbase, npts)], idx_s.at[pl.ds(0, npts), :])

            def st_rep(row, po, val):
                if is_tail:
                    a = row * npts + po
                    rep_s[a >> 7, pl.ds(a & (_SLAB - 1), _L)] = val
                else:
                    rep_s[row, pl.ds(po, _L)] = val

            def ld_rep(row, po):
                if is_tail:
                    a = row * npts + po
                    return rep_s[a >> 7, pl.ds(a & (_SLAB - 1), _L)]
                return rep_s[row, pl.ds(po, _L)]

            def st_exp(row, po, val):
                if is_tail:
                    a = row * npts + po
                    exp_s[a >> 7, pl.ds(a & (_SLAB - 1), _L)] = val
                else:
                    exp_s[row, pl.ds(po, _L)] = val

            for pg in range(npts // _L):
                base = sbase + pg * _L
                po = pg * _L
                xi = x_s[pl.ds(base, _L)]
                yi = y_s[pl.ds(base, _L)]
                zi = z_s[pl.ds(base, _L)]
                lane_po = lane + po

                def pass1(kk, carry):
                    maxd, sx, sy, sz, kkv = carry
                    iv = plsc.load_gather(idx_s, [lane_po, kkv])
                    nx = plsc.load_gather(x_s, [iv])
                    ny = plsc.load_gather(y_s, [iv])
                    nz = plsc.load_gather(z_s, [iv])
                    rx = xi - nx
                    ry = yi - ny
                    rz = zi - nz
                    xy2 = rx * rx + ry * ry
                    d2 = xy2 + rz * rz
                    xyd = _sqrt(xy2)
                    dis = _sqrt(d2)
                    alpha = _atan2(ry, rx)
                    beta = _atan2(rz, xyd, x_nonneg=True)
                    st_rep(kk, po, alpha)
                    st_rep(k + kk, po, beta)
                    st_rep(2 * k + kk, po, dis)
                    st_rep(3 * k + kk, po, xi)
                    st_rep(4 * k + kk, po, yi)
                    st_rep(5 * k + kk, po, zi)
                    st_rep(6 * k + kk, po, nx)
                    st_rep(7 * k + kk, po, ny)
                    st_rep(8 * k + kk, po, nz)
                    st_exp(kk, po, jnp.exp(-dis))
                    return (
                        jnp.maximum(maxd, dis),
                        sx + nx,
                        sy + ny,
                        sz + nz,
                        kkv + 1,
                    )

                zero = jnp.zeros((_L,), jnp.float32)
                maxd, sx, sy, sz, _ = plsc.parallel_loop(
                    0,
                    k,
                    unroll=3,
                    carry=(zero, zero, zero, zero, jnp.zeros((_L,), jnp.int32)),
                )(pass1)

                inv_k = 1.0 / k
                dx = xi - sx * inv_k
                dy = yi - sy * inv_k
                dz = zi - sz * inv_k
                dal = _atan2(dy, dx)
                dbe = _atan2(dz, _sqrt(dx * dx + dy * dy), x_nonneg=True)
                gd = _sqrt(xi * xi + yi * yi + zi * zi)
                lg_s[pl.ds(po, _L)] = maxd * maxd * maxd / gd
                dir_s[0, pl.ds(po, _L)] = dal
                dir_s[1, pl.ds(po, _L)] = dbe

            if not is_tail:
                h1 = pltpu.async_copy(
                    rep_s.at[pl.ds(2 * k, 7 * k), :],
                    o_rep.at[pl.ds(2 * k, 7 * k), pl.ds(sbase, _SLAB)],
                    sem,
                )
                h2 = pltpu.async_copy(
                    exp_s, o_exp.at[:, 0, pl.ds(sbase, _SLAB)], sem
                )
                h3 = pltpu.async_copy(lg_s, o_lg.at[pl.ds(sbase, _SLAB)], sem)

            for pg in range(npts // _L):
                po = pg * _L
                dal = dir_s[0, pl.ds(po, _L)]
                dbe = dir_s[1, pl.ds(po, _L)]

                @plsc.parallel_loop(0, k, unroll=2)
                def pass2(kk):
                    st_rep(kk, po, ld_rep(kk, po) - dal)
                    st_rep(k + kk, po, ld_rep(k + kk, po) - dbe)

            if not is_tail:
                h4 = pltpu.async_copy(
                    rep_s.at[pl.ds(0, 2 * k), :],
                    o_rep.at[pl.ds(0, 2 * k), pl.ds(sbase, _SLAB)],
                    sem,
                )
                h1.wait()
                h2.wait()
                h3.wait()
                h4.wait()
            else:
                nrows_r = 9 * k * npts // _SLAB
                nrows_e = k * npts // _SLAB
                pltpu.sync_copy(rep_s.at[pl.ds(0, nrows_r), :], o_rep_t)
                pltpu.sync_copy(exp_s.at[pl.ds(0, nrows_e), :], o_exp_t)
                pltpu.sync_copy(
                    lg_s.at[pl.ds(0, tail_pts)], o_lg.at[pl.ds(sbase, tail_pts)]
                )

        def slab_body(j, _):
            t = j * _NW + w

            @pl.when(t < full_slabs)
            def _():
                do_slab(t, _SLAB)

            if tail_pts:
                @pl.when(t == full_slabs)
                def _():
                    do_slab(t, tail_pts)

            return 0

        lax.fori_loop(0, j_iters, slab_body, 0)

    return sc_kernel


def kernel(xyz, neigh_idx, lengths):
    del lengths
    n, k = neigh_idx.shape
    stage = n + _L
    cols = jnp.zeros((3, stage), jnp.float32).at[:, :n].set(xyz.T).reshape(-1)
    rep, expd, lg, rep_t, exp_t = _make_sc_kernel(n, k)(cols, neigh_idx)
    big = rep.reshape(9, k, n).transpose(2, 1, 0)
    big_e = expd.transpose(2, 0, 1)
    tail = n % _SLAB
    if tail:
        nb = n - tail
        tail_rep = rep_t.reshape(9, k, tail).transpose(2, 1, 0)
        tail_exp = exp_t.reshape(k, tail).transpose(1, 0).reshape(tail, k, 1)
        big = lax.dynamic_update_slice(big, tail_rep, (nb, 0, 0))
        big_e = lax.dynamic_update_slice(big_e, tail_exp, (nb, 0, 0))
    return (big, big_e, lg.reshape(n, 1))


# speedup vs baseline: 1.1147x; 1.0081x over previous
import functools

import jax
import jax.numpy as jnp
from jax import lax
from jax.experimental import pallas as pl
from jax.experimental.pallas import tpu as pltpu
from jax.experimental.pallas import tpu_sc as plsc

_L = 16
_NW = 32
_SLAB = 128


def _rsqrt(x):
    i = lax.bitcast_convert_type(x, jnp.int32)
    y = lax.bitcast_convert_type(0x5F3759DF - (i >> 1), jnp.float32)
    xh = 0.5 * x
    y = y * (1.5 - xh * y * y)
    y = y * (1.5 - xh * y * y)
    return y


def _sqrt(x):
    return x * _rsqrt(x)


def _atan2(y, x, x_nonneg=False):
    ax = jnp.abs(x)
    ay = jnp.abs(y)
    hi = jnp.maximum(ax, ay)
    lo = jnp.minimum(ax, ay)
    t = lo / jnp.maximum(hi, 1e-30)
    z = t * t
    p = t * (
        0.99997726
        + z
        * (
            -0.33262347
            + z
            * (0.19354346 + z * (-0.11643287 + z * (0.05265332 + z * -0.01172120)))
        )
    )
    r = jnp.where(ay > ax, 1.5707963267949 - p, p)
    if not x_nonneg:
        r = jnp.where(x < 0.0, 3.14159265358979 - r, r)
    return jnp.where(y < 0.0, -r, r)


def _make_sc_kernel(n, k):
    full_slabs = n // _SLAB
    tail_pts = n % _SLAB
    assert tail_pts % _L == 0
    slabs = full_slabs + (1 if tail_pts else 0)
    j_iters = (slabs + _NW - 1) // _NW
    stage = n + _L

    mesh = plsc.VectorSubcoreMesh(core_axis_name="c", subcore_axis_name="s")
    out_type = (
        jax.ShapeDtypeStruct((9 * k, n), jnp.float32),
        jax.ShapeDtypeStruct((k, 1, n), jnp.float32),
        jax.ShapeDtypeStruct((n,), jnp.float32),
        jax.ShapeDtypeStruct((9 * k * max(tail_pts, _L) // _SLAB, _SLAB), jnp.float32),
        jax.ShapeDtypeStruct((k * max(tail_pts, _L) // _SLAB, _SLAB), jnp.float32),
    )
    scratch_types = [
        pltpu.VMEM((stage,), jnp.float32),
        pltpu.VMEM((stage,), jnp.float32),
        pltpu.VMEM((stage,), jnp.float32),
        pltpu.VMEM((_SLAB, k), jnp.int32),
        pltpu.VMEM((9 * k, _SLAB), jnp.float32),
        pltpu.VMEM((k, _SLAB), jnp.float32),
        pltpu.VMEM((_SLAB,), jnp.float32),
        pltpu.VMEM((2, _SLAB), jnp.float32),
        pltpu.SemaphoreType.DMA,
    ]

    @functools.partial(
        pl.kernel,
        out_type=out_type,
        mesh=mesh,
        scratch_types=scratch_types,
        compiler_params=pltpu.CompilerParams(needs_layout_passes=False),
    )
    def sc_kernel(cols_hbm, idx_hbm, o_rep, o_exp, o_lg, o_rep_t, o_exp_t,
                  x_s, y_s, z_s, idx_s, rep_s, exp_s, lg_s, dir_s, sem):
        w = lax.axis_index("s") * 2 + lax.axis_index("c")
        pltpu.sync_copy(cols_hbm.at[pl.ds(0, stage)], x_s)
        pltpu.sync_copy(cols_hbm.at[pl.ds(stage, stage)], y_s)
        pltpu.sync_copy(cols_hbm.at[pl.ds(2 * stage, stage)], z_s)

        lane = lax.iota(jnp.int32, _L)

        def do_slab(t, npts):
            is_tail = npts != _SLAB
            sbase = t * _SLAB
            pltpu.sync_copy(idx_hbm.at[pl.ds(sbase, npts)], idx_s.at[pl.ds(0, npts), :])

            def st_rep(row, po, val):
                if is_tail:
                    a = row * npts + po
                    rep_s[a >> 7, pl.ds(a & (_SLAB - 1), _L)] = val
                else:
                    rep_s[row, pl.ds(po, _L)] = val

            def ld_rep(row, po):
                if is_tail:
                    a = row * npts + po
                    return rep_s[a >> 7, pl.ds(a & (_SLAB - 1), _L)]
                return rep_s[row, pl.ds(po, _L)]

            def st_exp(row, po, val):
                if is_tail:
                    a = row * npts + po
                    exp_s[a >> 7, pl.ds(a & (_SLAB - 1), _L)] = val
                else:
                    exp_s[row, pl.ds(po, _L)] = val

            for pg in range(npts // _L):
                base = sbase + pg * _L
                po = pg * _L
                xi = x_s[pl.ds(base, _L)]
                yi = y_s[pl.ds(base, _L)]
                zi = z_s[pl.ds(base, _L)]
                lane_po = lane + po

                def pass1(kk, carry):
                    maxd, sx, sy, sz, kkv = carry
                    iv = plsc.load_gather(idx_s, [lane_po, kkv])
                    nx = plsc.load_gather(x_s, [iv])
                    ny = plsc.load_gather(y_s, [iv])
                    nz = plsc.load_gather(z_s, [iv])
                    rx = xi - nx
                    ry = yi - ny
                    rz = zi - nz
                    xy2 = rx * rx + ry * ry
                    d2 = xy2 + rz * rz
                    xyd = _sqrt(xy2)
                    dis = _sqrt(d2)
                    alpha = _atan2(ry, rx)
                    beta = _atan2(rz, xyd, x_nonneg=True)
                    st_rep(kk, po, alpha)
                    st_rep(k + kk, po, beta)
                    st_rep(2 * k + kk, po, dis)
                    st_rep(3 * k + kk, po, xi)
                    st_rep(4 * k + kk, po, yi)
                    st_rep(5 * k + kk, po, zi)
                    st_rep(6 * k + kk, po, nx)
                    st_rep(7 * k + kk, po, ny)
                    st_rep(8 * k + kk, po, nz)
                    st_exp(kk, po, jnp.exp(-dis))
                    return (
                        jnp.maximum(maxd, dis),
                        sx + nx,
                        sy + ny,
                        sz + nz,
                        kkv + 1,
                    )

                zero = jnp.zeros((_L,), jnp.float32)
                maxd, sx, sy, sz, _ = plsc.parallel_loop(
                    0,
                    k,
                    unroll=4,
                    carry=(zero, zero, zero, zero, jnp.zeros((_L,), jnp.int32)),
                )(pass1)

                inv_k = 1.0 / k
                dx = xi - sx * inv_k
                dy = yi - sy * inv_k
                dz = zi - sz * inv_k
                dal = _atan2(dy, dx)
                dbe = _atan2(dz, _sqrt(dx * dx + dy * dy), x_nonneg=True)
                gd = _sqrt(xi * xi + yi * yi + zi * zi)
                lg_s[pl.ds(po, _L)] = maxd * maxd * maxd / gd
                dir_s[0, pl.ds(po, _L)] = dal
                dir_s[1, pl.ds(po, _L)] = dbe

            if not is_tail:
                h1 = pltpu.async_copy(
                    rep_s.at[pl.ds(2 * k, 7 * k), :],
                    o_rep.at[pl.ds(2 * k, 7 * k), pl.ds(sbase, _SLAB)],
                    sem,
                )
                h2 = pltpu.async_copy(
                    exp_s, o_exp.at[:, 0, pl.ds(sbase, _SLAB)], sem
                )
                h3 = pltpu.async_copy(lg_s, o_lg.at[pl.ds(sbase, _SLAB)], sem)

            for pg in range(npts // _L):
                po = pg * _L
                dal = dir_s[0, pl.ds(po, _L)]
                dbe = dir_s[1, pl.ds(po, _L)]

                @plsc.parallel_loop(0, k, unroll=2)
                def pass2(kk):
                    st_rep(kk, po, ld_rep(kk, po) - dal)
                    st_rep(k + kk, po, ld_rep(k + kk, po) - dbe)

            if not is_tail:
                h4 = pltpu.async_copy(
                    rep_s.at[pl.ds(0, 2 * k), :],
                    o_rep.at[pl.ds(0, 2 * k), pl.ds(sbase, _SLAB)],
                    sem,
                )
                h1.wait()
                h2.wait()
                h3.wait()
                h4.wait()
            else:
                nrows_r = 9 * k * npts // _SLAB
                nrows_e = k * npts // _SLAB
                pltpu.sync_copy(rep_s.at[pl.ds(0, nrows_r), :], o_rep_t)
                pltpu.sync_copy(exp_s.at[pl.ds(0, nrows_e), :], o_exp_t)
                pltpu.sync_copy(
                    lg_s.at[pl.ds(0, tail_pts)], o_lg.at[pl.ds(sbase, tail_pts)]
                )

        def slab_body(j, _):
            t = j * _NW + w

            @pl.when(t < full_slabs)
            def _():
                do_slab(t, _SLAB)

            if tail_pts:
                @pl.when(t == full_slabs)
                def _():
                    do_slab(t, tail_pts)

            return 0

        lax.fori_loop(0, j_iters, slab_body, 0)

    return sc_kernel


def kernel(xyz, neigh_idx, lengths):
    del lengths
    n, k = neigh_idx.shape
    stage = n + _L
    cols = jnp.zeros((3, stage), jnp.float32).at[:, :n].set(xyz.T).reshape(-1)
    rep, expd, lg, rep_t, exp_t = _make_sc_kernel(n, k)(cols, neigh_idx)
    big = rep.reshape(9, k, n).transpose(2, 1, 0)
    big_e = expd.transpose(2, 0, 1)
    tail = n % _SLAB
    if tail:
        nb = n - tail
        tail_rep = rep_t.reshape(9, k, tail).transpose(2, 1, 0)
        tail_exp = exp_t.reshape(k, tail).transpose(1, 0).reshape(tail, k, 1)
        big = lax.dynamic_update_slice(big, tail_rep, (nb, 0, 0))
        big_e = lax.dynamic_update_slice(big_e, tail_exp, (nb, 0, 0))
    return (big, big_e, lg.reshape(n, 1))
